# Initial kernel scaffold; baseline (speedup 1.0000x reference)
import jax
import jax.numpy as jnp
from jax.experimental import pallas as pl


def kernel(x, router_w, corr_bias, gate_w, up_w, down_w, shared_gate_w, shared_up_w, shared_down_w):
    raise NotImplementedError("write your pallas kernel here")


# speedup vs baseline: 1.3984x; 1.3984x over previous
import functools

import jax
import jax.numpy as jnp
from jax import lax
from jax.experimental import pallas as pl
from jax.experimental.pallas import tpu as pltpu
from jax.experimental.pallas import tpu_sc as plsc

E = 16; G = 4; EPG = 4; K = 2; TOPK_GROUP = 2
C = 2048; H = 1024; H_SHARED = 1024

BLK = 256
NEG_INF = float("-inf")


def _num_blocks(S):
    SK = S * K
    return -(-(SK + E * (BLK - 1)) // BLK)


def _router_shared_body(x_ref, rw_ref, rb_ref, sg_ref, su_ref, sd_ref,
                        shared_ref, topk_ref, fw_ref):
    xb = x_ref[...]

    g = jax.lax.dot_general(xb, sg_ref[...], (((1,), (1,)), ((), ())),
                            preferred_element_type=jnp.float32)
    u = jax.lax.dot_general(xb, su_ref[...], (((1,), (1,)), ((), ())),
                            preferred_element_type=jnp.float32)
    h = (g * jax.nn.sigmoid(g)) * u
    sh = jax.lax.dot_general(h, sd_ref[...], (((1,), (1,)), ((), ())),
                             preferred_element_type=jnp.float32)
    shared_ref[...] = sh

    logits = jax.lax.dot_general(xb, rw_ref[...], (((1,), (1,)), ((), ())),
                                 preferred_element_type=jnp.float32)
    scores = jax.nn.sigmoid(logits)
    sb = scores + rb_ref[...]

    TS = xb.shape[0]
    le = lax.broadcasted_iota(jnp.int32, (TS, E), 1)
    gl = le // EPG

    gcols = []
    for g_id in range(G):
        gm = jnp.max(jnp.where(gl == g_id, sb, NEG_INF), axis=1, keepdims=True)
        gcols.append(gm)
    gsc = jnp.concatenate(gcols, axis=1)

    gi = lax.broadcasted_iota(jnp.int32, (TS, G), 1)
    m1 = jnp.max(gsc, axis=1, keepdims=True)
    g1 = jnp.min(jnp.where(gsc == m1, gi, G), axis=1, keepdims=True)
    gsc2 = jnp.where(gi == g1, NEG_INF, gsc)
    m2 = jnp.max(gsc2, axis=1, keepdims=True)
    g2 = jnp.min(jnp.where(gsc2 == m2, gi, G), axis=1, keepdims=True)

    gmask = (gl == g1) | (gl == g2)
    msb = jnp.where(gmask, sb, NEG_INF)

    v1 = jnp.max(msb, axis=1, keepdims=True)
    e1 = jnp.min(jnp.where(msb == v1, le, E), axis=1, keepdims=True)
    msb2 = jnp.where(le == e1, NEG_INF, msb)
    v2 = jnp.max(msb2, axis=1, keepdims=True)
    e2 = jnp.min(jnp.where(msb2 == v2, le, E), axis=1, keepdims=True)

    w1 = jnp.sum(jnp.where(le == e1, scores, 0.0), axis=1, keepdims=True)
    w2 = jnp.sum(jnp.where(le == e2, scores, 0.0), axis=1, keepdims=True)
    norm = w1 + w2 + 1e-20

    topk_ref[...] = jnp.concatenate([e1, e2], axis=1)
    fw_ref[...] = jnp.concatenate([w1 / norm, w2 / norm], axis=1)


def _router_shared(xf, router_w, corr_bias, sgw, suw, sdw):
    S = xf.shape[0]
    TS = 256
    grid = (S // TS,)
    return pl.pallas_call(
        _router_shared_body,
        grid=grid,
        in_specs=[
            pl.BlockSpec((TS, C), lambda i: (i, 0)),
            pl.BlockSpec((E, C), lambda i: (0, 0)),
            pl.BlockSpec((1, E), lambda i: (0, 0)),
            pl.BlockSpec((H_SHARED, C), lambda i: (0, 0)),
            pl.BlockSpec((H_SHARED, C), lambda i: (0, 0)),
            pl.BlockSpec((C, H_SHARED), lambda i: (0, 0)),
        ],
        out_specs=[
            pl.BlockSpec((TS, C), lambda i: (i, 0)),
            pl.BlockSpec((TS, K), lambda i: (i, 0)),
            pl.BlockSpec((TS, K), lambda i: (i, 0)),
        ],
        out_shape=[
            jax.ShapeDtypeStruct((S, C), jnp.float32),
            jax.ShapeDtypeStruct((S, K), jnp.int32),
            jax.ShapeDtypeStruct((S, K), jnp.float32),
        ],
    )(xf, router_w, corr_bias.reshape(1, E), sgw, suw, sdw)


def _dispatch_metadata(topk, fw, S):
    SK = S * K
    NB = _num_blocks(S)
    P = NB * BLK

    e_flat = topk.reshape(SK)
    w_flat = fw.reshape(SK)
    order = jnp.argsort(e_flat, stable=True)
    e_sorted = e_flat[order]

    counts = jnp.zeros((E,), jnp.int32).at[e_flat].add(1)
    offset = jnp.concatenate([jnp.zeros((1,), jnp.int32),
                              jnp.cumsum(counts)[:-1].astype(jnp.int32)])
    padded = -(-counts // BLK) * BLK
    pad_end = jnp.cumsum(padded).astype(jnp.int32)
    pad_start = jnp.concatenate([jnp.zeros((1,), jnp.int32), pad_end[:-1]])

    j = jnp.arange(SK, dtype=jnp.int32)
    ppos = pad_start[e_sorted] + (j - offset[e_sorted])

    pos = jnp.zeros((SK,), jnp.int32).at[order].set(ppos).reshape(S, K)
    tids_pad = jnp.zeros((P,), jnp.int32).at[ppos].set(order // K)
    w_pad = jnp.zeros((P,), jnp.float32).at[ppos].set(w_flat[order])

    bstart = jnp.arange(NB, dtype=jnp.int32) * BLK
    beid = jnp.searchsorted(pad_end, bstart, side="right").astype(jnp.int32)
    beidc = jnp.minimum(beid, E - 1)
    valid = ((beid < E) &
             ((bstart - pad_start[beidc]) < counts[beidc])).astype(jnp.int32)
    beid_f = lax.associative_scan(jnp.maximum, jnp.where(valid == 1, beidc, 0))
    return (beid_f.astype(jnp.int32), valid, tids_pad,
            w_pad.reshape(P, 1), pos[:, 0], pos[:, 1], NB, P)


def _sc_gather(xf, tids_pad, P):
    info = plsc.get_sparse_core_info()
    NW = info.num_cores * info.num_subcores
    rows_per_w = P // NW
    CH = 16
    n_ch = rows_per_w // CH
    mesh = plsc.VectorSubcoreMesh(core_axis_name="c", subcore_axis_name="s")

    @functools.partial(
        pl.kernel, mesh=mesh,
        out_type=jax.ShapeDtypeStruct((P, C), jnp.float32),
        scratch_types=[
            pltpu.VMEM((CH,), jnp.int32),
            pltpu.VMEM((CH, C), jnp.float32),
            pltpu.SemaphoreType.DMA,
        ],
    )
    def k(x_hbm, idx_hbm, out_hbm, idx_v, rows_v, sem):
        wid = lax.axis_index("s") * info.num_cores + lax.axis_index("c")
        base = wid * rows_per_w

        def body(c, _):
            b = base + c * CH
            pltpu.sync_copy(idx_hbm.at[pl.ds(b, CH)], idx_v)
            pltpu.async_copy(x_hbm.at[idx_v], rows_v, sem).wait()
            pltpu.sync_copy(rows_v, out_hbm.at[pl.ds(b, CH)])
            return _

        lax.fori_loop(0, n_ch, body, 0)

    return k(xf, tids_pad)


def _grouped_body(eids_ref, valid_ref, xg_ref, gw_ref, uw_ref, dw_ref, w_ref,
                  out_ref):
    b = pl.program_id(0)

    @pl.when(valid_ref[b] == 1)
    def _():
        xb = xg_ref[...]
        g = jnp.dot(xb, gw_ref[0], preferred_element_type=jnp.float32)
        u = jnp.dot(xb, uw_ref[0], preferred_element_type=jnp.float32)
        h = (g * jax.nn.sigmoid(g)) * u
        y = jnp.dot(h, dw_ref[0], preferred_element_type=jnp.float32)
        out_ref[...] = y * w_ref[...]


def _grouped_mm(xg, gate_w, up_w, down_w, w_pad, eids, valid, NB, P):
    grid_spec = pltpu.PrefetchScalarGridSpec(
        num_scalar_prefetch=2,
        grid=(NB,),
        in_specs=[
            pl.BlockSpec((BLK, C), lambda i, e, v: (i, 0)),
            pl.BlockSpec((1, C, H), lambda i, e, v: (e[i], 0, 0)),
            pl.BlockSpec((1, C, H), lambda i, e, v: (e[i], 0, 0)),
            pl.BlockSpec((1, H, C), lambda i, e, v: (e[i], 0, 0)),
            pl.BlockSpec((BLK, 1), lambda i, e, v: (i, 0)),
        ],
        out_specs=pl.BlockSpec((BLK, C), lambda i, e, v: (i, 0)),
    )
    return pl.pallas_call(
        _grouped_body,
        grid_spec=grid_spec,
        out_shape=jax.ShapeDtypeStruct((P, C), jnp.float32),
    )(eids, valid, xg, gate_w, up_w, down_w, w_pad)


def _sc_combine(shared, yg, pos0, pos1, S):
    info = plsc.get_sparse_core_info()
    NW = info.num_cores * info.num_subcores
    rows_per_w = S // NW
    CH = 8
    n_ch = rows_per_w // CH
    LP = C // 16
    mesh = plsc.VectorSubcoreMesh(core_axis_name="c", subcore_axis_name="s")

    @functools.partial(
        pl.kernel, mesh=mesh,
        out_type=jax.ShapeDtypeStruct((S, C), jnp.float32),
        scratch_types=[
            pltpu.VMEM((CH,), jnp.int32),
            pltpu.VMEM((CH,), jnp.int32),
            pltpu.VMEM((CH, C), jnp.float32),
            pltpu.VMEM((CH, C), jnp.float32),
            pltpu.VMEM((CH, C), jnp.float32),
            pltpu.SemaphoreType.DMA,
            pltpu.SemaphoreType.DMA,
        ],
    )
    def k(sh_hbm, yg_hbm, p0_hbm, p1_hbm, out_hbm,
          i0_v, i1_v, a_v, b_v, s_v, sem0, sem1):
        wid = lax.axis_index("s") * info.num_cores + lax.axis_index("c")
        base = wid * rows_per_w

        def body(c, _):
            b = base + c * CH
            pltpu.sync_copy(p0_hbm.at[pl.ds(b, CH)], i0_v)
            pltpu.sync_copy(p1_hbm.at[pl.ds(b, CH)], i1_v)
            cp0 = pltpu.async_copy(yg_hbm.at[i0_v], a_v, sem0)
            cp1 = pltpu.async_copy(yg_hbm.at[i1_v], b_v, sem1)
            pltpu.sync_copy(sh_hbm.at[pl.ds(b, CH)], s_v)
            cp0.wait()
            cp1.wait()

            def add_row(r, _2):
                def add_vec(jj, _3):
                    sl = pl.ds(jj * 16, 16)
                    s_v[r, sl] = s_v[r, sl] + a_v[r, sl] + b_v[r, sl]
                    return _3
                lax.fori_loop(0, LP, add_vec, 0)
                return _2

            lax.fori_loop(0, CH, add_row, 0)
            pltpu.sync_copy(s_v, out_hbm.at[pl.ds(b, CH)])
            return _

        lax.fori_loop(0, n_ch, body, 0)

    return k(shared, yg, pos0, pos1)


def kernel(x, router_w, corr_bias, gate_w, up_w, down_w,
           shared_gate_w, shared_up_w, shared_down_w):
    Bx, Tx, Cx = x.shape
    S = Bx * Tx
    xf = x.reshape(S, Cx)

    shared, topk, fw = _router_shared(xf, router_w, corr_bias,
                                      shared_gate_w, shared_up_w,
                                      shared_down_w)
    (eids, valid, tids_pad, w_pad, pos0, pos1, NB, P) = _dispatch_metadata(
        topk, fw, S)

    xg = _sc_gather(xf, tids_pad, P)
    yg = _grouped_mm(xg, gate_w, up_w, down_w, w_pad, eids, valid, NB, P)
    out = _sc_combine(shared, yg, pos0, pos1, S)
    return out.reshape(Bx, Tx, Cx)


# speedup vs baseline: 1.9612x; 1.4024x over previous
import functools

import jax
import jax.numpy as jnp
from jax import lax
from jax.experimental import pallas as pl
from jax.experimental.pallas import tpu as pltpu
from jax.experimental.pallas import tpu_sc as plsc

E = 16; G = 4; EPG = 4; K = 2; TOPK_GROUP = 2
C = 2048; H = 1024; H_SHARED = 1024

BLK = 256
NEG_INF = float("-inf")


def _num_blocks(S):
    SK = S * K
    return -(-(SK + E * (BLK - 1)) // BLK)


def _router_body(x_ref, rw_ref, rb_ref, topk_ref, fw_ref):
    xb = x_ref[...]
    logits = jax.lax.dot_general(xb, rw_ref[...], (((1,), (1,)), ((), ())),
                                 preferred_element_type=jnp.float32)
    scores = jax.nn.sigmoid(logits)
    sb = scores + rb_ref[...]

    TS = xb.shape[0]
    le = lax.broadcasted_iota(jnp.int32, (TS, E), 1)
    gl = le // EPG

    gcols = []
    for g_id in range(G):
        gm = jnp.max(jnp.where(gl == g_id, sb, NEG_INF), axis=1, keepdims=True)
        gcols.append(gm)
    gsc = jnp.concatenate(gcols, axis=1)

    gi = lax.broadcasted_iota(jnp.int32, (TS, G), 1)
    m1 = jnp.max(gsc, axis=1, keepdims=True)
    g1 = jnp.min(jnp.where(gsc == m1, gi, G), axis=1, keepdims=True)
    gsc2 = jnp.where(gi == g1, NEG_INF, gsc)
    m2 = jnp.max(gsc2, axis=1, keepdims=True)
    g2 = jnp.min(jnp.where(gsc2 == m2, gi, G), axis=1, keepdims=True)

    gmask = (gl == g1) | (gl == g2)
    msb = jnp.where(gmask, sb, NEG_INF)

    v1 = jnp.max(msb, axis=1, keepdims=True)
    e1 = jnp.min(jnp.where(msb == v1, le, E), axis=1, keepdims=True)
    msb2 = jnp.where(le == e1, NEG_INF, msb)
    v2 = jnp.max(msb2, axis=1, keepdims=True)
    e2 = jnp.min(jnp.where(msb2 == v2, le, E), axis=1, keepdims=True)

    w1 = jnp.sum(jnp.where(le == e1, scores, 0.0), axis=1, keepdims=True)
    w2 = jnp.sum(jnp.where(le == e2, scores, 0.0), axis=1, keepdims=True)
    norm = w1 + w2 + 1e-20

    topk_ref[...] = jnp.concatenate([e1, e2], axis=1)
    fw_ref[...] = jnp.concatenate([w1 / norm, w2 / norm], axis=1)


def _router(xf, router_w, corr_bias):
    S = xf.shape[0]
    TS = 512
    return pl.pallas_call(
        _router_body,
        grid=(S // TS,),
        in_specs=[
            pl.BlockSpec((TS, C), lambda i: (i, 0)),
            pl.BlockSpec((E, C), lambda i: (0, 0)),
            pl.BlockSpec((1, E), lambda i: (0, 0)),
        ],
        out_specs=[
            pl.BlockSpec((TS, K), lambda i: (i, 0)),
            pl.BlockSpec((TS, K), lambda i: (i, 0)),
        ],
        out_shape=[
            jax.ShapeDtypeStruct((S, K), jnp.int32),
            jax.ShapeDtypeStruct((S, K), jnp.float32),
        ],
    )(xf, router_w, corr_bias.reshape(1, E))


def _shared_body(x_ref, sg_ref, su_ref, sd_ref, out_ref):
    xb = x_ref[...]
    g = jax.lax.dot_general(xb, sg_ref[...], (((1,), (1,)), ((), ())),
                            preferred_element_type=jnp.float32)
    u = jax.lax.dot_general(xb, su_ref[...], (((1,), (1,)), ((), ())),
                            preferred_element_type=jnp.float32)
    h = (g * jax.nn.sigmoid(g)) * u
    out_ref[...] = jax.lax.dot_general(h, sd_ref[...], (((1,), (1,)), ((), ())),
                                       preferred_element_type=jnp.float32)


def _shared_expert(xf, sgw, suw, sdw):
    S = xf.shape[0]
    TS = 256
    return pl.pallas_call(
        _shared_body,
        grid=(S // TS,),
        in_specs=[
            pl.BlockSpec((TS, C), lambda i: (i, 0)),
            pl.BlockSpec((H_SHARED, C), lambda i: (0, 0)),
            pl.BlockSpec((H_SHARED, C), lambda i: (0, 0)),
            pl.BlockSpec((C, H_SHARED), lambda i: (0, 0)),
        ],
        out_specs=pl.BlockSpec((TS, C), lambda i: (i, 0)),
        out_shape=jax.ShapeDtypeStruct((S, C), jnp.float32),
    )(xf, sgw, suw, sdw)


def _dispatch_metadata(topk, fw, S):
    SK = S * K
    NB = _num_blocks(S)
    P = NB * BLK

    e_flat = topk.reshape(SK)
    w_flat = fw.reshape(SK)
    order = jnp.argsort(e_flat, stable=True)
    e_sorted = e_flat[order]

    counts = jnp.zeros((E,), jnp.int32).at[e_flat].add(1)
    offset = jnp.concatenate([jnp.zeros((1,), jnp.int32),
                              jnp.cumsum(counts)[:-1].astype(jnp.int32)])
    padded = -(-counts // BLK) * BLK
    pad_end = jnp.cumsum(padded).astype(jnp.int32)
    pad_start = jnp.concatenate([jnp.zeros((1,), jnp.int32), pad_end[:-1]])

    j = jnp.arange(SK, dtype=jnp.int32)
    ppos = pad_start[e_sorted] + (j - offset[e_sorted])

    pos = jnp.zeros((SK,), jnp.int32).at[order].set(ppos).reshape(S, K)
    tids_pad = jnp.zeros((P,), jnp.int32).at[ppos].set(order // K)
    w_pad = jnp.zeros((P,), jnp.float32).at[ppos].set(w_flat[order])

    bstart = jnp.arange(NB, dtype=jnp.int32) * BLK
    beid = jnp.searchsorted(pad_end, bstart, side="right").astype(jnp.int32)
    beidc = jnp.minimum(beid, E - 1)
    valid = ((beid < E) &
             ((bstart - pad_start[beidc]) < counts[beidc])).astype(jnp.int32)
    beid_f = lax.associative_scan(jnp.maximum, jnp.where(valid == 1, beidc, 0))
    total_vec = jnp.full((16,), pad_end[-1], dtype=jnp.int32)
    pos_inter = pos.reshape(SK)
    return (beid_f.astype(jnp.int32), valid, tids_pad,
            w_pad.reshape(P, 1), pos_inter, total_vec, NB, P)


def _sc_gather(xf, tids_pad, total_vec, P):
    info = plsc.get_sparse_core_info()
    NW = info.num_cores * info.num_subcores
    rows_per_w = P // NW
    CH = 16
    max_ch = rows_per_w // CH
    mesh = plsc.VectorSubcoreMesh(core_axis_name="c", subcore_axis_name="s")

    @functools.partial(
        pl.kernel, mesh=mesh,
        out_type=jax.ShapeDtypeStruct((P, C), jnp.float32),
        compiler_params=pltpu.CompilerParams(needs_layout_passes=False),
        scratch_types=[
            pltpu.VMEM((16,), jnp.int32),
            pltpu.VMEM((CH,), jnp.int32),
            pltpu.VMEM((CH,), jnp.int32),
            pltpu.VMEM((CH, C), jnp.float32),
            pltpu.VMEM((CH, C), jnp.float32),
            pltpu.SemaphoreType.DMA,
            pltpu.SemaphoreType.DMA,
            pltpu.SemaphoreType.DMA,
            pltpu.SemaphoreType.DMA,
        ],
    )
    def k(x_hbm, idx_hbm, tot_hbm, out_hbm,
          tot_v, i0_v, i1_v, r0_v, r1_v, g0, g1, w0, w1):
        wid = lax.axis_index("s") * info.num_cores + lax.axis_index("c")
        base = wid * rows_per_w

        pltpu.sync_copy(tot_hbm, tot_v)
        total = jnp.max(tot_v[...], axis=0)
        need = jnp.clip(total - base, 0, rows_per_w)
        n_ch = (need + CH - 1) // CH

        ivs = (i0_v, i1_v); rvs = (r0_v, r1_v)
        gsems = (g0, g1); wsems = (w0, w1)

        def start(c, slot):
            b = base + c * CH
            pltpu.sync_copy(idx_hbm.at[pl.ds(b, CH)], ivs[slot])
            pltpu.async_copy(x_hbm.at[ivs[slot]], rvs[slot], gsems[slot])

        def finish(c, slot):
            pltpu.make_async_copy(x_hbm.at[ivs[slot]], rvs[slot],
                                  gsems[slot]).wait()
            b = base + c * CH
            pltpu.async_copy(rvs[slot], out_hbm.at[pl.ds(b, CH)], wsems[slot])

        def wb_wait(c, slot):
            b = base + c * CH
            pltpu.make_async_copy(rvs[slot], out_hbm.at[pl.ds(b, CH)],
                                  wsems[slot]).wait()

        @pl.when(0 < n_ch)
        def _():
            start(0, 0)
        for c in range(max_ch):
            sl = c % 2
            nsl = (c + 1) % 2
            if c + 1 < max_ch:
                @pl.when(c + 1 < n_ch)
                def _(c=c, nsl=nsl):
                    if c >= 1:
                        wb_wait(c - 1, nsl)
                    start(c + 1, nsl)

            @pl.when(c < n_ch)
            def _(c=c, sl=sl):
                finish(c, sl)
        for c in range(max_ch):
            @pl.when((c == n_ch - 2) | (c == n_ch - 1))
            def _(c=c):
                wb_wait(c, c % 2)

    return k(xf, tids_pad, total_vec)


def _grouped_body(eids_ref, valid_ref, xg_ref, gw_ref, uw_ref, dw_ref, w_ref,
                  out_ref):
    b = pl.program_id(0)

    @pl.when(valid_ref[b] == 1)
    def _():
        xb = xg_ref[...]
        g = jnp.dot(xb, gw_ref[0], preferred_element_type=jnp.float32)
        u = jnp.dot(xb, uw_ref[0], preferred_element_type=jnp.float32)
        h = (g * jax.nn.sigmoid(g)) * u
        y = jnp.dot(h, dw_ref[0], preferred_element_type=jnp.float32)
        out_ref[...] = y * w_ref[...]


def _grouped_mm(xg, gate_w, up_w, down_w, w_pad, eids, valid, NB, P):
    grid_spec = pltpu.PrefetchScalarGridSpec(
        num_scalar_prefetch=2,
        grid=(NB,),
        in_specs=[
            pl.BlockSpec((BLK, C), lambda i, e, v: (i, 0)),
            pl.BlockSpec((1, C, H), lambda i, e, v: (e[i], 0, 0)),
            pl.BlockSpec((1, C, H), lambda i, e, v: (e[i], 0, 0)),
            pl.BlockSpec((1, H, C), lambda i, e, v: (e[i], 0, 0)),
            pl.BlockSpec((BLK, 1), lambda i, e, v: (i, 0)),
        ],
        out_specs=pl.BlockSpec((BLK, C), lambda i, e, v: (i, 0)),
    )
    return pl.pallas_call(
        _grouped_body,
        grid_spec=grid_spec,
        out_shape=jax.ShapeDtypeStruct((P, C), jnp.float32),
    )(eids, valid, xg, gate_w, up_w, down_w, w_pad)


def _sc_combine(shared, yg, pos_inter, S):
    info = plsc.get_sparse_core_info()
    NW = info.num_cores * info.num_subcores
    rows_per_w = S // NW
    CH = 8
    n_ch = rows_per_w // CH
    mesh = plsc.VectorSubcoreMesh(core_axis_name="c", subcore_axis_name="s")

    @functools.partial(
        pl.kernel, mesh=mesh,
        out_type=jax.ShapeDtypeStruct((S, C), jnp.float32),
        scratch_types=[
            pltpu.VMEM((2 * CH,), jnp.int32),
            pltpu.VMEM((2 * CH, C), jnp.float32),
            pltpu.VMEM((CH, C), jnp.float32),
            pltpu.VMEM((2 * CH,), jnp.int32),
            pltpu.VMEM((2 * CH, C), jnp.float32),
            pltpu.VMEM((CH, C), jnp.float32),
            pltpu.SemaphoreType.DMA,
            pltpu.SemaphoreType.DMA,
            pltpu.SemaphoreType.DMA,
            pltpu.SemaphoreType.DMA,
        ],
    )
    def k(sh_hbm, yg_hbm, pi_hbm, out_hbm,
          ia, ra, sa, ib, rb, sb_, ga, sha, gb, shb):
        wid = lax.axis_index("s") * info.num_cores + lax.axis_index("c")
        base = wid * rows_per_w

        ivs = (ia, ib); rvs = (ra, rb); svs = (sa, sb_)
        gsems = (ga, gb); ssems = (sha, shb)

        def start(c, slot):
            b = base + c * CH
            pltpu.sync_copy(pi_hbm.at[pl.ds(2 * b, 2 * CH)], ivs[slot])
            pltpu.async_copy(yg_hbm.at[ivs[slot]], rvs[slot], gsems[slot])
            pltpu.async_copy(sh_hbm.at[pl.ds(b, CH)], svs[slot], ssems[slot])

        def finish(c, slot):
            b = base + c * CH
            pltpu.make_async_copy(yg_hbm.at[ivs[slot]], rvs[slot],
                                  gsems[slot]).wait()
            pltpu.make_async_copy(sh_hbm.at[pl.ds(b, CH)], svs[slot],
                                  ssems[slot]).wait()
            sv = svs[slot]; rv = rvs[slot]

            def add_row(r, _2):
                def add_grp(j, _3):
                    for u in range(8):
                        sl = pl.ds(j * 128 + u * 16, 16)
                        sv[r, sl] = sv[r, sl] + rv[2 * r, sl] + rv[2 * r + 1, sl]
                    return _3
                lax.fori_loop(0, C // 128, add_grp, 0)
                return _2

            lax.fori_loop(0, CH, add_row, 0)
            pltpu.async_copy(sv, out_hbm.at[pl.ds(b, CH)], ssems[slot])

        def wb_wait(c, slot):
            b = base + c * CH
            pltpu.make_async_copy(svs[slot], out_hbm.at[pl.ds(b, CH)],
                                  ssems[slot]).wait()

        start(0, 0)
        for c in range(n_ch):
            if c + 1 < n_ch:
                if c >= 1:
                    wb_wait(c - 1, (c + 1) % 2)
                start(c + 1, (c + 1) % 2)
            finish(c, c % 2)
        if n_ch >= 2:
            wb_wait(n_ch - 2, n_ch % 2)
        wb_wait(n_ch - 1, (n_ch - 1) % 2)

    return k(shared, yg, pos_inter)


def kernel(x, router_w, corr_bias, gate_w, up_w, down_w,
           shared_gate_w, shared_up_w, shared_down_w):
    Bx, Tx, Cx = x.shape
    S = Bx * Tx
    xf = x.reshape(S, Cx)

    topk, fw = _router(xf, router_w, corr_bias)
    (eids, valid, tids_pad, w_pad, pos_inter, total_vec, NB, P) = \
        _dispatch_metadata(topk, fw, S)

    xg = _sc_gather(xf, tids_pad, total_vec, P)
    shared = _shared_expert(xf, shared_gate_w, shared_up_w, shared_down_w)
    yg = _grouped_mm(xg, gate_w, up_w, down_w, w_pad, eids, valid, NB, P)
    out = _sc_combine(shared, yg, pos_inter, S)
    return out.reshape(Bx, Tx, Cx)
